# baseline (speedup 1.0000x reference)
import jax
import jax.numpy as jnp
from jax import lax
from jax.experimental import pallas as pl
from jax.experimental.pallas import tpu as pltpu
from jax.experimental.pallas import tpu_sc as plsc

_N = 100000
_E = 1600000
_EMB = 32
_Q = 8
_OUT = 2

_NW = 32
_CH = 128
_J = 8
_M = 52
_EPW = _CH * _J * _M
_EPAD = _NW * _EPW
_ROWS_PER_SUB = 12512
_R = 16 * _ROWS_PER_SUB
_DUMMY = 2 * _N

_B1 = 2000
_B4 = 2000
_EB = 12500


def _leaky(v):
  return jnp.where(v >= 0, v, 0.01 * v)


def _encoder_body(des_ref, tweet_ref, num_ref, cat_ref,
                  wd_ref, bd_ref, wt_ref, bt_ref,
                  wn_ref, bn_ref, wc_ref, bc_ref,
                  g_ref, y_ref):
  d = _leaky(jnp.dot(des_ref[...], wd_ref[...],
                     preferred_element_type=jnp.float32) + bd_ref[...])
  t = _leaky(jnp.dot(tweet_ref[...], wt_ref[...],
                     preferred_element_type=jnp.float32) + bt_ref[...])
  n = _leaky(jnp.dot(num_ref[...], wn_ref[...],
                     preferred_element_type=jnp.float32) + bn_ref[...])
  c = _leaky(jnp.dot(cat_ref[...], wc_ref[...],
                     preferred_element_type=jnp.float32) + bc_ref[...])
  x = jnp.concatenate((d, t, n, c), axis=1)
  y = jnp.dot(x, g_ref[...], preferred_element_type=jnp.float32)
  col = lax.broadcasted_iota(jnp.int32, y.shape, 1)
  y_ref[...] = y + (col == 6).astype(jnp.float32)


def _bidx_body(dst_ref, typ_ref, bidx_ref):
  bidx_ref[...] = dst_ref[...] + _N * typ_ref[...]


def _edge_body(y_hbm, src_hbm, bidx_hbm, zeros_hbm, out_hbm,
               idx_v, bidx_v, rows_v, acc_sh, gsem, ssem, isem):
  c = lax.axis_index("c")
  s = lax.axis_index("s")
  wid = s * 2 + c

  z0 = s * _ROWS_PER_SUB
  pltpu.sync_copy(zeros_hbm.at[pl.ds(z0, _ROWS_PER_SUB)],
                  acc_sh.at[pl.ds(z0, _ROWS_PER_SUB)])
  plsc.subcore_barrier()

  base = wid * (_J * _M)

  def stage(g, slot):
    row0 = base + g * _J
    return [
        pltpu.async_copy(src_hbm.at[pl.ds(row0, _J)],
                         idx_v.at[pl.ds(slot * _J, _J)], isem),
        pltpu.async_copy(bidx_hbm.at[pl.ds(row0, _J)],
                         bidx_v.at[pl.ds(slot * _J, _J)], isem),
    ]

  def gather_pairs(slot):
    return [(y_hbm.at[idx_v.at[slot * _J + j]], rows_v.at[slot, j])
            for j in range(_J)]

  def scatter_pairs(slot):
    return [(rows_v.at[slot, j], acc_sh.at[bidx_v.at[slot * _J + j]])
            for j in range(_J)]

  def iter_group(g, slot, first=False, last=False):
    if not first:
      for src, dst in scatter_pairs(1 - slot):
        pltpu.make_async_copy(src, dst, ssem).wait()
    if not last:
      staged = stage(g + 1, 1 - slot)
    for src, dst in gather_pairs(slot):
      pltpu.make_async_copy(src, dst, gsem).wait()
    for src, dst in scatter_pairs(slot):
      pltpu.async_copy(src, dst, ssem, add=True)
    if not last:
      for d in staged:
        d.wait()
      for src, dst in gather_pairs(1 - slot):
        pltpu.async_copy(src, dst, gsem)

  for d in stage(0, 0):
    d.wait()
  for src, dst in gather_pairs(0):
    pltpu.async_copy(src, dst, gsem)
  iter_group(0, 0, first=True)

  @pl.loop(1, _M - 1, step=2)
  def _(g0):
    iter_group(g0, 1)
    iter_group(g0 + 1, 0)

  iter_group(_M - 1, 1, last=True)
  for src, dst in scatter_pairs(1):
    pltpu.make_async_copy(src, dst, ssem).wait()

  plsc.subcore_barrier()
  pltpu.sync_copy(acc_sh.at[pl.ds(z0, _ROWS_PER_SUB)],
                  out_hbm.at[c, pl.ds(z0, _ROWS_PER_SUB)])


def _combine_body(y_ref, a00_ref, a01_ref, a10_ref, a11_ref, out_ref):
  r0 = a00_ref[0] + a10_ref[0]
  r1 = a01_ref[0] + a11_ref[0]
  cnt0 = r0[:, 6:7]
  cnt1 = r1[:, 6:7]
  mean0 = jnp.where(cnt0 > 0, r0[:, 0:2] / jnp.maximum(cnt0, 1.0), 0.0)
  mean1 = jnp.where(cnt1 > 0, r1[:, 2:4] / jnp.maximum(cnt1, 1.0), 0.0)
  out_ref[...] = y_ref[:, 4:6] + mean0 + mean1


def kernel(des, tweet, num_prop, cat_prop, edge_index, edge_type,
           W_des, b_des, W_tweet, b_tweet, W_num, b_num, W_cat, b_cat,
           W_rel, W_root):
  f32 = jnp.float32

  G = jnp.concatenate(
      (W_rel[0], W_rel[1], W_root, jnp.zeros((_EMB, 2), f32)), axis=1)

  grid1 = _N // _B1
  full = lambda i: (0, 0)
  y_table = pl.pallas_call(
      _encoder_body,
      grid=(grid1,),
      in_specs=[
          pl.BlockSpec((_B1, 768), lambda i: (i, 0)),
          pl.BlockSpec((_B1, 768), lambda i: (i, 0)),
          pl.BlockSpec((_B1, 5), lambda i: (i, 0)),
          pl.BlockSpec((_B1, 1), lambda i: (i, 0)),
          pl.BlockSpec((768, _Q), full),
          pl.BlockSpec((1, _Q), full),
          pl.BlockSpec((768, _Q), full),
          pl.BlockSpec((1, _Q), full),
          pl.BlockSpec((5, _Q), full),
          pl.BlockSpec((1, _Q), full),
          pl.BlockSpec((1, _Q), full),
          pl.BlockSpec((1, _Q), full),
          pl.BlockSpec((_EMB, 8), full),
      ],
      out_specs=pl.BlockSpec((_B1, 8), lambda i: (i, 0)),
      out_shape=jax.ShapeDtypeStruct((_N, 8), f32),
  )(des, tweet, num_prop, cat_prop,
    W_des, b_des.reshape(1, _Q), W_tweet, b_tweet.reshape(1, _Q),
    W_num, b_num.reshape(1, _Q), W_cat, b_cat.reshape(1, _Q), G)

  dst2d = edge_index[1].reshape(_E // _CH, _CH)
  typ2d = edge_type.reshape(_E // _CH, _CH)
  bidx2d = pl.pallas_call(
      _bidx_body,
      grid=(_E // _CH // _EB,),
      in_specs=[
          pl.BlockSpec((_EB, _CH), lambda i: (i, 0)),
          pl.BlockSpec((_EB, _CH), lambda i: (i, 0)),
      ],
      out_specs=pl.BlockSpec((_EB, _CH), lambda i: (i, 0)),
      out_shape=jax.ShapeDtypeStruct((_E // _CH, _CH), jnp.int32),
  )(dst2d, typ2d)

  pad = _EPAD - _E
  src_p = jnp.concatenate(
      (edge_index[0], jnp.zeros((pad,), jnp.int32))).reshape(_EPAD // _CH, _CH)
  bidx_p = jnp.concatenate(
      (bidx2d.reshape(_E), jnp.full((pad,), _DUMMY, jnp.int32))
  ).reshape(_EPAD // _CH, _CH)
  zeros_rows = jnp.zeros((_R, 8), f32)

  mesh = plsc.VectorSubcoreMesh(core_axis_name="c", subcore_axis_name="s")
  acc = pl.kernel(
      _edge_body,
      out_type=jax.ShapeDtypeStruct((2, _R, 8), f32),
      mesh=mesh,
      scratch_types=[
          pltpu.VMEM((2 * _J, _CH), jnp.int32),
          pltpu.VMEM((2 * _J, _CH), jnp.int32),
          pltpu.VMEM((2, _J, _CH, 8), f32),
          pltpu.VMEM_SHARED((_R, 8), f32),
          pltpu.SemaphoreType.DMA,
          pltpu.SemaphoreType.DMA,
          pltpu.SemaphoreType.DMA,
      ],
      compiler_params=pltpu.CompilerParams(use_tc_tiling_on_sc=False),
  )(y_table, src_p, bidx_p, zeros_rows)

  grid4 = _N // _B4
  nblk = _N // _B4

  def acc_spec(coff, roff):
    return pl.BlockSpec(
        (1, _B4, 8), lambda i, _c=coff, _r=roff: (_c, _r + i, 0))

  out = pl.pallas_call(
      _combine_body,
      grid=(grid4,),
      in_specs=[
          pl.BlockSpec((_B4, 8), lambda i: (i, 0)),
          acc_spec(0, 0),
          acc_spec(0, nblk),
          acc_spec(1, 0),
          acc_spec(1, nblk),
      ],
      out_specs=pl.BlockSpec((_B4, _OUT), lambda i: (i, 0)),
      out_shape=jax.ShapeDtypeStruct((_N, _OUT), f32),
  )(y_table, acc, acc, acc, acc)
  return out


# speedup vs baseline: 1.0851x; 1.0851x over previous
import jax
import jax.numpy as jnp
from jax import lax
from jax.experimental import pallas as pl
from jax.experimental.pallas import tpu as pltpu
from jax.experimental.pallas import tpu_sc as plsc

_N = 100000
_E = 1600000
_EMB = 32
_Q = 8
_OUT = 2

_NW = 32
_CH = 128
_J = 24
_M = 17
_EPW = _CH * _J * _M
_EPAD = _NW * _EPW
_ROWS_PER_SUB = 12512
_R = 16 * _ROWS_PER_SUB
_DUMMY = 2 * _N

_B1 = 2000
_B4 = 2000
_EB = 12500


def _leaky(v):
  return jnp.where(v >= 0, v, 0.01 * v)


def _encoder_body(des_ref, tweet_ref, num_ref, cat_ref,
                  wd_ref, bd_ref, wt_ref, bt_ref,
                  wn_ref, bn_ref, wc_ref, bc_ref,
                  g_ref, y_ref):
  d = _leaky(jnp.dot(des_ref[...], wd_ref[...],
                     preferred_element_type=jnp.float32) + bd_ref[...])
  t = _leaky(jnp.dot(tweet_ref[...], wt_ref[...],
                     preferred_element_type=jnp.float32) + bt_ref[...])
  n = _leaky(jnp.dot(num_ref[...], wn_ref[...],
                     preferred_element_type=jnp.float32) + bn_ref[...])
  c = _leaky(jnp.dot(cat_ref[...], wc_ref[...],
                     preferred_element_type=jnp.float32) + bc_ref[...])
  x = jnp.concatenate((d, t, n, c), axis=1)
  y = jnp.dot(x, g_ref[...], preferred_element_type=jnp.float32)
  col = lax.broadcasted_iota(jnp.int32, y.shape, 1)
  y_ref[...] = y + (col == 6).astype(jnp.float32)


def _bidx_body(dst_ref, typ_ref, bidx_ref):
  bidx_ref[...] = dst_ref[...] + _N * typ_ref[...]


def _edge_body(y_hbm, src_hbm, bidx_hbm, zeros_hbm, out_hbm,
               idx_v, bidx_v, rows_v, acc_sh, gsem, ssem):
  c = lax.axis_index("c")
  s = lax.axis_index("s")
  wid = s * 2 + c

  z0 = s * _ROWS_PER_SUB
  pltpu.sync_copy(zeros_hbm.at[pl.ds(z0, _ROWS_PER_SUB)],
                  acc_sh.at[pl.ds(z0, _ROWS_PER_SUB)])
  plsc.subcore_barrier()

  rows_per_worker = _EPW // _CH

  def step(i, carry):
    row0 = wid * rows_per_worker + i * _J
    pltpu.sync_copy(src_hbm.at[pl.ds(row0, _J)], idx_v)
    pltpu.sync_copy(bidx_hbm.at[pl.ds(row0, _J)], bidx_v)
    gathers = [
        pltpu.async_copy(y_hbm.at[idx_v.at[j]], rows_v.at[j], gsem)
        for j in range(_J)
    ]
    for g in gathers:
      g.wait()
    scatters = [
        pltpu.async_copy(rows_v.at[j], acc_sh.at[bidx_v.at[j]], ssem, add=True)
        for j in range(_J)
    ]
    for sc in scatters:
      sc.wait()
    return carry

  lax.fori_loop(0, _M, step, 0)

  plsc.subcore_barrier()
  pltpu.sync_copy(acc_sh.at[pl.ds(z0, _ROWS_PER_SUB)],
                  out_hbm.at[c, pl.ds(z0, _ROWS_PER_SUB)])


def _combine_body(y_ref, a00_ref, a01_ref, a10_ref, a11_ref, out_ref):
  r0 = a00_ref[0] + a10_ref[0]
  r1 = a01_ref[0] + a11_ref[0]
  cnt0 = r0[:, 6:7]
  cnt1 = r1[:, 6:7]
  mean0 = jnp.where(cnt0 > 0, r0[:, 0:2] / jnp.maximum(cnt0, 1.0), 0.0)
  mean1 = jnp.where(cnt1 > 0, r1[:, 2:4] / jnp.maximum(cnt1, 1.0), 0.0)
  out_ref[...] = y_ref[:, 4:6] + mean0 + mean1


def kernel(des, tweet, num_prop, cat_prop, edge_index, edge_type,
           W_des, b_des, W_tweet, b_tweet, W_num, b_num, W_cat, b_cat,
           W_rel, W_root):
  f32 = jnp.float32

  G = jnp.concatenate(
      (W_rel[0], W_rel[1], W_root, jnp.zeros((_EMB, 2), f32)), axis=1)

  grid1 = _N // _B1
  full = lambda i: (0, 0)
  y_table = pl.pallas_call(
      _encoder_body,
      grid=(grid1,),
      in_specs=[
          pl.BlockSpec((_B1, 768), lambda i: (i, 0)),
          pl.BlockSpec((_B1, 768), lambda i: (i, 0)),
          pl.BlockSpec((_B1, 5), lambda i: (i, 0)),
          pl.BlockSpec((_B1, 1), lambda i: (i, 0)),
          pl.BlockSpec((768, _Q), full),
          pl.BlockSpec((1, _Q), full),
          pl.BlockSpec((768, _Q), full),
          pl.BlockSpec((1, _Q), full),
          pl.BlockSpec((5, _Q), full),
          pl.BlockSpec((1, _Q), full),
          pl.BlockSpec((1, _Q), full),
          pl.BlockSpec((1, _Q), full),
          pl.BlockSpec((_EMB, 8), full),
      ],
      out_specs=pl.BlockSpec((_B1, 8), lambda i: (i, 0)),
      out_shape=jax.ShapeDtypeStruct((_N, 8), f32),
  )(des, tweet, num_prop, cat_prop,
    W_des, b_des.reshape(1, _Q), W_tweet, b_tweet.reshape(1, _Q),
    W_num, b_num.reshape(1, _Q), W_cat, b_cat.reshape(1, _Q), G)

  dst2d = edge_index[1].reshape(_E // _CH, _CH)
  typ2d = edge_type.reshape(_E // _CH, _CH)
  bidx2d = pl.pallas_call(
      _bidx_body,
      grid=(_E // _CH // _EB,),
      in_specs=[
          pl.BlockSpec((_EB, _CH), lambda i: (i, 0)),
          pl.BlockSpec((_EB, _CH), lambda i: (i, 0)),
      ],
      out_specs=pl.BlockSpec((_EB, _CH), lambda i: (i, 0)),
      out_shape=jax.ShapeDtypeStruct((_E // _CH, _CH), jnp.int32),
  )(dst2d, typ2d)

  pad = _EPAD - _E
  src_p = jnp.concatenate(
      (edge_index[0], jnp.zeros((pad,), jnp.int32))).reshape(_EPAD // _CH, _CH)
  bidx_p = jnp.concatenate(
      (bidx2d.reshape(_E), jnp.full((pad,), _DUMMY, jnp.int32))
  ).reshape(_EPAD // _CH, _CH)
  zeros_rows = jnp.zeros((_R, 8), f32)

  mesh = plsc.VectorSubcoreMesh(core_axis_name="c", subcore_axis_name="s")
  acc = pl.kernel(
      _edge_body,
      out_type=jax.ShapeDtypeStruct((2, _R, 8), f32),
      mesh=mesh,
      scratch_types=[
          pltpu.VMEM((_J, _CH), jnp.int32),
          pltpu.VMEM((_J, _CH), jnp.int32),
          pltpu.VMEM((_J, _CH, 8), f32),
          pltpu.VMEM_SHARED((_R, 8), f32),
          pltpu.SemaphoreType.DMA,
          pltpu.SemaphoreType.DMA,
      ],
      compiler_params=pltpu.CompilerParams(use_tc_tiling_on_sc=False),
  )(y_table, src_p, bidx_p, zeros_rows)

  grid4 = _N // _B4
  nblk = _N // _B4

  def acc_spec(coff, roff):
    return pl.BlockSpec(
        (1, _B4, 8), lambda i, _c=coff, _r=roff: (_c, _r + i, 0))

  out = pl.pallas_call(
      _combine_body,
      grid=(grid4,),
      in_specs=[
          pl.BlockSpec((_B4, 8), lambda i: (i, 0)),
          acc_spec(0, 0),
          acc_spec(0, nblk),
          acc_spec(1, 0),
          acc_spec(1, nblk),
      ],
      out_specs=pl.BlockSpec((_B4, _OUT), lambda i: (i, 0)),
      out_shape=jax.ShapeDtypeStruct((_N, _OUT), f32),
  )(y_table, acc, acc, acc, acc)
  return out


# speedup vs baseline: 1.2452x; 1.1476x over previous
import jax
import jax.numpy as jnp
from jax import lax
from jax.experimental import pallas as pl
from jax.experimental.pallas import tpu as pltpu
from jax.experimental.pallas import tpu_sc as plsc

_N = 100000
_E = 1600000
_EMB = 32
_Q = 8
_OUT = 2

_NW = 32
_CH = 128
_J = 16
_M = 25
_EPW = _CH * _J * _M
_EPAD = _NW * _EPW
_ROWS_PER_SUB = 12512
_R = 16 * _ROWS_PER_SUB
_DUMMY = 2 * _N

_SB = 4

_B1 = 2000
_B4 = 2000
_EB = 12500


def _leaky(v):
  return jnp.where(v >= 0, v, 0.01 * v)


def _encoder_body(des_ref, tweet_ref, num_ref, cat_ref,
                  wd_ref, bd_ref, wt_ref, bt_ref,
                  wn_ref, bn_ref, wc_ref, bc_ref,
                  g_ref, y_ref):
  d = _leaky(jnp.dot(des_ref[...], wd_ref[...],
                     preferred_element_type=jnp.float32) + bd_ref[...])
  t = _leaky(jnp.dot(tweet_ref[...], wt_ref[...],
                     preferred_element_type=jnp.float32) + bt_ref[...])
  n = _leaky(jnp.dot(num_ref[...], wn_ref[...],
                     preferred_element_type=jnp.float32) + bn_ref[...])
  c = _leaky(jnp.dot(cat_ref[...], wc_ref[...],
                     preferred_element_type=jnp.float32) + bc_ref[...])
  x = jnp.concatenate((d, t, n, c), axis=1)
  y = jnp.dot(x, g_ref[...], preferred_element_type=jnp.float32)
  col = lax.broadcasted_iota(jnp.int32, y.shape, 1)
  y_ref[...] = y + (col == 6).astype(jnp.float32)


def _bidx_body(dst_ref, typ_ref, bidx_ref):
  bidx_ref[...] = dst_ref[...] + _N * typ_ref[...]


def _edge_body(y_hbm, src_hbm, bidx_hbm, zeros_hbm, out_hbm,
               idx_v, bidx_v, rows_v, acc_sh, gsem, ssem, stsem):
  c = lax.axis_index("c")
  s = lax.axis_index("s")
  wid = s * 2 + c

  z0 = s * _ROWS_PER_SUB
  pltpu.sync_copy(zeros_hbm.at[pl.ds(z0, _ROWS_PER_SUB)],
                  acc_sh.at[pl.ds(z0, _ROWS_PER_SUB)])
  plsc.subcore_barrier()

  rows_per_worker = _EPW // _CH
  js = _J // _SB

  def step(i, carry):
    row0 = wid * rows_per_worker + i * _J
    st1 = pltpu.async_copy(src_hbm.at[pl.ds(row0, _J)], idx_v, stsem)
    st2 = pltpu.async_copy(bidx_hbm.at[pl.ds(row0, _J)], bidx_v, stsem)
    st1.wait()
    st2.wait()
    gathers = {0: [pltpu.async_copy(y_hbm.at[idx_v.at[j]], rows_v.at[j], gsem)
                   for j in range(js)]}
    scatters = []
    for b in range(_SB):
      for g in gathers[b]:
        g.wait()
      if b + 1 < _SB:
        gathers[b + 1] = [
            pltpu.async_copy(y_hbm.at[idx_v.at[j]], rows_v.at[j], gsem)
            for j in range((b + 1) * js, (b + 2) * js)
        ]
      scatters += [
          pltpu.async_copy(rows_v.at[j], acc_sh.at[bidx_v.at[j]], ssem,
                           add=True)
          for j in range(b * js, (b + 1) * js)
      ]
    for sc in scatters:
      sc.wait()
    return carry

  lax.fori_loop(0, _M, step, 0)

  plsc.subcore_barrier()
  pltpu.sync_copy(acc_sh.at[pl.ds(z0, _ROWS_PER_SUB)],
                  out_hbm.at[c, pl.ds(z0, _ROWS_PER_SUB)])


def _combine_body(y_ref, a00_ref, a01_ref, a10_ref, a11_ref, out_ref):
  r0 = a00_ref[0] + a10_ref[0]
  r1 = a01_ref[0] + a11_ref[0]
  cnt0 = r0[:, 6:7]
  cnt1 = r1[:, 6:7]
  mean0 = jnp.where(cnt0 > 0, r0[:, 0:2] / jnp.maximum(cnt0, 1.0), 0.0)
  mean1 = jnp.where(cnt1 > 0, r1[:, 2:4] / jnp.maximum(cnt1, 1.0), 0.0)
  out_ref[...] = y_ref[:, 4:6] + mean0 + mean1


def kernel(des, tweet, num_prop, cat_prop, edge_index, edge_type,
           W_des, b_des, W_tweet, b_tweet, W_num, b_num, W_cat, b_cat,
           W_rel, W_root):
  f32 = jnp.float32

  G = jnp.concatenate(
      (W_rel[0], W_rel[1], W_root, jnp.zeros((_EMB, 2), f32)), axis=1)

  grid1 = _N // _B1
  full = lambda i: (0, 0)
  y_table = pl.pallas_call(
      _encoder_body,
      grid=(grid1,),
      in_specs=[
          pl.BlockSpec((_B1, 768), lambda i: (i, 0)),
          pl.BlockSpec((_B1, 768), lambda i: (i, 0)),
          pl.BlockSpec((_B1, 5), lambda i: (i, 0)),
          pl.BlockSpec((_B1, 1), lambda i: (i, 0)),
          pl.BlockSpec((768, _Q), full),
          pl.BlockSpec((1, _Q), full),
          pl.BlockSpec((768, _Q), full),
          pl.BlockSpec((1, _Q), full),
          pl.BlockSpec((5, _Q), full),
          pl.BlockSpec((1, _Q), full),
          pl.BlockSpec((1, _Q), full),
          pl.BlockSpec((1, _Q), full),
          pl.BlockSpec((_EMB, 8), full),
      ],
      out_specs=pl.BlockSpec((_B1, 8), lambda i: (i, 0)),
      out_shape=jax.ShapeDtypeStruct((_N, 8), f32),
  )(des, tweet, num_prop, cat_prop,
    W_des, b_des.reshape(1, _Q), W_tweet, b_tweet.reshape(1, _Q),
    W_num, b_num.reshape(1, _Q), W_cat, b_cat.reshape(1, _Q), G)

  dst2d = edge_index[1].reshape(_E // _CH, _CH)
  typ2d = edge_type.reshape(_E // _CH, _CH)
  bidx2d = pl.pallas_call(
      _bidx_body,
      grid=(_E // _CH // _EB,),
      in_specs=[
          pl.BlockSpec((_EB, _CH), lambda i: (i, 0)),
          pl.BlockSpec((_EB, _CH), lambda i: (i, 0)),
      ],
      out_specs=pl.BlockSpec((_EB, _CH), lambda i: (i, 0)),
      out_shape=jax.ShapeDtypeStruct((_E // _CH, _CH), jnp.int32),
  )(dst2d, typ2d)

  pad = _EPAD - _E
  src_p = jnp.concatenate(
      (edge_index[0], jnp.zeros((pad,), jnp.int32))).reshape(_EPAD // _CH, _CH)
  bidx_p = jnp.concatenate(
      (bidx2d.reshape(_E), jnp.full((pad,), _DUMMY, jnp.int32))
  ).reshape(_EPAD // _CH, _CH)
  zeros_rows = jnp.zeros((_R, 8), f32)

  mesh = plsc.VectorSubcoreMesh(core_axis_name="c", subcore_axis_name="s")
  acc = pl.kernel(
      _edge_body,
      out_type=jax.ShapeDtypeStruct((2, _R, 8), f32),
      mesh=mesh,
      scratch_types=[
          pltpu.VMEM((_J, _CH), jnp.int32),
          pltpu.VMEM((_J, _CH), jnp.int32),
          pltpu.VMEM((_J, _CH, 8), f32),
          pltpu.VMEM_SHARED((_R, 8), f32),
          pltpu.SemaphoreType.DMA,
          pltpu.SemaphoreType.DMA,
          pltpu.SemaphoreType.DMA,
      ],
      compiler_params=pltpu.CompilerParams(use_tc_tiling_on_sc=False),
  )(y_table, src_p, bidx_p, zeros_rows)

  grid4 = _N // _B4
  nblk = _N // _B4

  def acc_spec(coff, roff):
    return pl.BlockSpec(
        (1, _B4, 8), lambda i, _c=coff, _r=roff: (_c, _r + i, 0))

  out = pl.pallas_call(
      _combine_body,
      grid=(grid4,),
      in_specs=[
          pl.BlockSpec((_B4, 8), lambda i: (i, 0)),
          acc_spec(0, 0),
          acc_spec(0, nblk),
          acc_spec(1, 0),
          acc_spec(1, nblk),
      ],
      out_specs=pl.BlockSpec((_B4, _OUT), lambda i: (i, 0)),
      out_shape=jax.ShapeDtypeStruct((_N, _OUT), f32),
  )(y_table, acc, acc, acc, acc)
  return out
